# baseline (speedup 1.0000x reference)
import jax
import jax.numpy as jnp
from jax import lax
from jax.experimental import pallas as pl
from jax.experimental.pallas import tpu as pltpu
from jax.experimental.pallas import tpu_sc as plsc

_N = 10000
_E = 320000
_D = 128
_DE = 16
_H = 256
_EPS = 1e-7
_BN_EPS = 1e-5

_NC = 2
_NS = 16
_LANES = 16
_NW = _NC * _NS
_EPW = _E // _NW
_BLK = 80
_NBLK = _EPW // _BLK
_NPAD = 10240
_RPT = _NPAD // _NS
_ZR = 128
_CHUNKS = _D // _LANES

_MESH = dict(core_axis_name="c", subcore_axis_name="s",
             num_cores=_NC, num_subcores=_NS)


def _sc_msg_body(x_hbm, e_hbm, src_hbm, dst_hbm, out_hbm,
                 agg_sh, zrow_v, src_v, dst_v, xrows_v, erows_v, sem):
    cid = lax.axis_index("c")
    sid = lax.axis_index("s")
    wid = cid * _NS + sid

    def _zrow(r, _):
        for c in range(_CHUNKS):
            zrow_v[r, pl.ds(c * _LANES, _LANES)] = jnp.zeros((_LANES,), jnp.float32)
        return 0
    lax.fori_loop(0, _ZR, _zrow, 0)

    row0 = sid * _RPT
    for k in range(_RPT // _ZR):
        pltpu.sync_copy(zrow_v, agg_sh.at[pl.ds(row0 + k * _ZR, _ZR)])
    plsc.subcore_barrier()

    def _block(blk, _):
        base = wid * _EPW + blk * _BLK
        pltpu.sync_copy(src_hbm.at[pl.ds(base, _BLK)], src_v)
        pltpu.sync_copy(dst_hbm.at[pl.ds(base, _BLK)], dst_v)
        pltpu.async_copy(x_hbm.at[src_v], xrows_v, sem).wait()
        pltpu.sync_copy(e_hbm.at[pl.ds(base, _BLK)], erows_v)

        def _row(r, _):
            for c in range(_CHUNKS):
                sl = pl.ds(c * _LANES, _LANES)
                v = xrows_v[r, sl] + erows_v[r, sl]
                xrows_v[r, sl] = jnp.maximum(v, 0.0) + _EPS
            return 0
        lax.fori_loop(0, _BLK, _row, 0)

        pltpu.sync_copy(xrows_v, agg_sh.at[dst_v], add=True)
        return 0
    lax.fori_loop(0, _NBLK, _block, 0)

    plsc.subcore_barrier()
    for k in range(_RPT // _ZR):
        r0 = row0 + k * _ZR
        pltpu.sync_copy(agg_sh.at[pl.ds(r0, _ZR)], out_hbm.at[cid, pl.ds(r0, _ZR)])


_sc_msg = pl.kernel(
    _sc_msg_body,
    out_type=jax.ShapeDtypeStruct((_NC, _NPAD, _D), jnp.float32),
    mesh=plsc.VectorSubcoreMesh(**_MESH),
    scratch_types=[
        pltpu.VMEM_SHARED((_NPAD, _D), jnp.float32),
        pltpu.VMEM((_ZR, _D), jnp.float32),
        pltpu.VMEM((_BLK,), jnp.int32),
        pltpu.VMEM((_BLK,), jnp.int32),
        pltpu.VMEM((_BLK, _D), jnp.float32),
        pltpu.VMEM((_BLK, _D), jnp.float32),
        pltpu.SemaphoreType.DMA,
    ],
    compiler_params=pltpu.CompilerParams(use_tc_tiling_on_sc=False),
    name="genconv_sc_msg",
)


def _sc_cnt_body(dst_hbm, cnt_hbm, cnt_sh, zcnt_v, dst_v, ones_v):
    cid = lax.axis_index("c")
    sid = lax.axis_index("s")
    wid = cid * _NS + sid

    def _zcnt(r, _):
        zcnt_v[r, :] = jnp.zeros((_LANES,), jnp.float32)
        return 0
    lax.fori_loop(0, _RPT, _zcnt, 0)

    def _ones(r, _):
        ones_v[r, :] = jnp.ones((_LANES,), jnp.float32)
        return 0
    lax.fori_loop(0, _BLK, _ones, 0)

    row0 = sid * _RPT
    pltpu.sync_copy(zcnt_v, cnt_sh.at[pl.ds(row0, _RPT)])
    plsc.subcore_barrier()

    def _block(blk, _):
        base = wid * _EPW + blk * _BLK
        pltpu.sync_copy(dst_hbm.at[pl.ds(base, _BLK)], dst_v)
        pltpu.sync_copy(ones_v, cnt_sh.at[dst_v], add=True)
        return 0
    lax.fori_loop(0, _NBLK, _block, 0)

    plsc.subcore_barrier()
    pltpu.sync_copy(cnt_sh.at[pl.ds(row0, _RPT)], cnt_hbm.at[cid, pl.ds(row0, _RPT)])


_sc_cnt = pl.kernel(
    _sc_cnt_body,
    out_type=jax.ShapeDtypeStruct((_NC, _NPAD, _LANES), jnp.float32),
    mesh=plsc.VectorSubcoreMesh(**_MESH),
    scratch_types=[
        pltpu.VMEM_SHARED((_NPAD, _LANES), jnp.float32),
        pltpu.VMEM((_RPT, _LANES), jnp.float32),
        pltpu.VMEM((_BLK,), jnp.int32),
        pltpu.VMEM((_BLK, _LANES), jnp.float32),
    ],
    compiler_params=pltpu.CompilerParams(use_tc_tiling_on_sc=False),
    name="genconv_sc_cnt",
)


_EB = 2000


def _edge_proj_body(ea_ref, w0_ref, w1_ref, w2_ref, e0_ref, e1_ref, e2_ref):
    a = ea_ref[...]
    e0_ref[...] = jnp.dot(a, w0_ref[...], preferred_element_type=jnp.float32)
    e1_ref[...] = jnp.dot(a, w1_ref[...], preferred_element_type=jnp.float32)
    e2_ref[...] = jnp.dot(a, w2_ref[...], preferred_element_type=jnp.float32)


def _edge_proj(edge_attr, We0, We1, We2):
    grid = (_E // _EB,)
    w_spec = pl.BlockSpec((_DE, _D), lambda i: (0, 0))
    e_spec = pl.BlockSpec((_EB, _D), lambda i: (i, 0))
    return pl.pallas_call(
        _edge_proj_body,
        grid=grid,
        in_specs=[pl.BlockSpec((_EB, _DE), lambda i: (i, 0)), w_spec, w_spec, w_spec],
        out_specs=[e_spec, e_spec, e_spec],
        out_shape=[jax.ShapeDtypeStruct((_E, _D), jnp.float32)] * 3,
    )(edge_attr, We0, We1, We2)


_NB = 2000


def _mlp_body(final, p0_ref, p1_ref, inv_ref, x_ref, w1_ref, w2_ref, g_ref,
              b_ref, wh_ref, bh_ref, o_ref):
    out = (p0_ref[0] + p1_ref[0]) * inv_ref[...] + x_ref[...]
    h = jnp.dot(out, w1_ref[...], preferred_element_type=jnp.float32)
    scale = g_ref[...] * (1.0 / jnp.sqrt(1.0 + _BN_EPS))
    h = jnp.maximum(h * scale + b_ref[...], 0.0)
    y = jnp.maximum(jnp.dot(h, w2_ref[...], preferred_element_type=jnp.float32), 0.0)
    if final:
        o_ref[...] = jnp.sum(y * wh_ref[...], axis=1, keepdims=True) + bh_ref[...]
    else:
        o_ref[...] = y


def _mlp(p, inv, x, W1, W2, gamma, beta, Wh, bh, final):
    import functools
    grid = (_N // _NB,)
    in_specs = [
        pl.BlockSpec((1, _NB, _D), lambda i: (0, i, 0)),
        pl.BlockSpec((1, _NB, _D), lambda i: (1, i, 0)),
        pl.BlockSpec((_NB, 1), lambda i: (i, 0)),
        pl.BlockSpec((_NB, _D), lambda i: (i, 0)),
        pl.BlockSpec((_D, _H), lambda i: (0, 0)),
        pl.BlockSpec((_H, _D), lambda i: (0, 0)),
        pl.BlockSpec((1, _H), lambda i: (0, 0)),
        pl.BlockSpec((1, _H), lambda i: (0, 0)),
        pl.BlockSpec((1, _D), lambda i: (0, 0)),
        pl.BlockSpec((1, 1), lambda i: (0, 0)),
    ]
    if final:
        out_spec = pl.BlockSpec((_NB, 1), lambda i: (i, 0))
        out_shape = jax.ShapeDtypeStruct((_N, 1), jnp.float32)
    else:
        out_spec = pl.BlockSpec((_NB, _D), lambda i: (i, 0))
        out_shape = jax.ShapeDtypeStruct((_N, _D), jnp.float32)
    return pl.pallas_call(
        functools.partial(_mlp_body, final),
        grid=grid,
        in_specs=in_specs,
        out_specs=out_spec,
        out_shape=out_shape,
    )(p, p, inv, x, W1, W2, gamma.reshape(1, _H), beta.reshape(1, _H),
      Wh.reshape(1, _D), bh.reshape(1, 1))


def kernel(x, edge_index, edge_attr, batch, num_graphs, graph_features,
           We0, W10, W20, gamma0, beta0,
           We1, W11, W21, gamma1, beta1,
           We2, W12, W22, gamma2, beta2,
           Wh, bh):
    src = edge_index[0]
    dst = edge_index[1]
    e0, e1, e2 = _edge_proj(edge_attr, We0, We1, We2)

    cnt2 = _sc_cnt(dst)
    cnt = cnt2[0, :_N, 0] + cnt2[1, :_N, 0]
    inv = (1.0 / jnp.maximum(cnt, 1.0)).reshape(_N, 1)

    p = _sc_msg(x, e0, src, dst)
    h1 = _mlp(p, inv, x, W10, W20, gamma0, beta0, Wh, bh, final=False)
    p = _sc_msg(h1, e1, src, dst)
    h2 = _mlp(p, inv, h1, W11, W21, gamma1, beta1, Wh, bh, final=False)
    p = _sc_msg(h2, e2, src, dst)
    return _mlp(p, inv, h2, W12, W22, gamma2, beta2, Wh, bh, final=True)


# speedup vs baseline: 1.9280x; 1.9280x over previous
import jax
import jax.numpy as jnp
from jax import lax
from jax.experimental import pallas as pl
from jax.experimental.pallas import tpu as pltpu
from jax.experimental.pallas import tpu_sc as plsc

_N = 10000
_E = 320000
_D = 128
_DE = 16
_H = 256
_EPS = 1e-7
_BN_EPS = 1e-5

_NC = 2
_NS = 16
_LANES = 16
_NW = _NC * _NS
_EPW = _E // _NW
_BLK = 40
_NBLK = _EPW // _BLK
_RPT = 640
_STRIDE = 624
_CPAD = 10240
_CRPT = _CPAD // _NS
_CHUNKS = _D // _LANES

_MESH = dict(core_axis_name="c", subcore_axis_name="s",
             num_cores=_NC, num_subcores=_NS)


def _sc_msg_body(x_hbm, e_hbm, z_hbm, src_hbm, dst3_hbm, out_hbm, agg_sh,
                 srcall_v, dstall_v,
                 xb0, xb1, mb0, mb1,
                 gs0, gs1, es0, es1, ss0, ss1):
    xb = (xb0, xb1)
    mb = (mb0, mb1)
    gs = (gs0, gs1)
    es = (es0, es1)
    ss = (ss0, ss1)
    cid = lax.axis_index("c")
    sid = lax.axis_index("s")
    wid = cid * _NS + sid
    ebase = wid * _EPW

    pltpu.sync_copy(src_hbm.at[pl.ds(ebase, _EPW)], srcall_v)
    pltpu.sync_copy(dst3_hbm.at[wid], dstall_v)

    row0 = jnp.minimum(sid * _STRIDE, _N - _RPT)
    pltpu.sync_copy(z_hbm.at[pl.ds(row0, _RPT)], agg_sh.at[pl.ds(row0, _RPT)])
    plsc.subcore_barrier()

    def _gather(k, b):
        return pltpu.make_async_copy(
            x_hbm.at[srcall_v.at[pl.ds(k * _BLK, _BLK)]], xb[b], gs[b])

    def _eload(k, b):
        return pltpu.make_async_copy(
            e_hbm.at[pl.ds(ebase + k * _BLK, _BLK)], mb[b], es[b])

    def _scat_start(k, b):
        pltpu.async_copy(mb[b], agg_sh.at[dstall_v.at[k]], ss[b], add=True)

    def _scat_wait(b):
        pltpu.make_async_copy(mb[b], agg_sh.at[dstall_v.at[0]], ss[b]).wait()

    _gather(0, 0).start()
    _gather(1, 1).start()
    _eload(0, 0).start()

    def _grp(g, _):
        for b in range(2):
            k = g * 2 + b
            bo = 1 - b

            @pl.when(k >= 1)
            def _():
                _scat_wait(bo)

            @pl.when(k + 1 < _NBLK)
            def _():
                _eload(k + 1, bo).start()

            _gather(k, b).wait()
            _eload(0, b).wait()

            def _row(r, _):
                for c in range(_CHUNKS):
                    sl = pl.ds(c * _LANES, _LANES)
                    v = xb[b][r, sl] + mb[b][r, sl]
                    mb[b][r, sl] = jnp.maximum(v, 0.0) + _EPS
                return 0
            lax.fori_loop(0, _BLK, _row, 0)

            _scat_start(k, b)

            @pl.when(k + 2 < _NBLK)
            def _():
                _gather(k + 2, b).start()
        return 0
    lax.fori_loop(0, _NBLK // 2, _grp, 0)

    _scat_wait(1)

    plsc.subcore_barrier()
    pltpu.sync_copy(agg_sh.at[pl.ds(row0, _RPT)], out_hbm.at[cid, pl.ds(row0, _RPT)])


_sc_msg = pl.kernel(
    _sc_msg_body,
    out_type=jax.ShapeDtypeStruct((_NC, _N, _D), jnp.float32),
    mesh=plsc.VectorSubcoreMesh(**_MESH),
    scratch_types=[
        pltpu.VMEM_SHARED((_N, _D), jnp.float32),
        pltpu.VMEM((_EPW,), jnp.int32),
        pltpu.VMEM((_NBLK, _BLK), jnp.int32),
    ] + [pltpu.VMEM((_BLK, _D), jnp.float32)] * 4
      + [pltpu.SemaphoreType.DMA] * 6,
    compiler_params=pltpu.CompilerParams(use_tc_tiling_on_sc=False),
    name="genconv_sc_msg",
)


def _sc_cnt_body(dst_hbm, cnt_hbm, cnt_sh, zcnt_v, dst_v, ones_v):
    cid = lax.axis_index("c")
    sid = lax.axis_index("s")
    wid = cid * _NS + sid

    def _zcnt(r, _):
        zcnt_v[r, :] = jnp.zeros((_LANES,), jnp.float32)
        return 0
    lax.fori_loop(0, _CRPT, _zcnt, 0)

    def _ones(r, _):
        ones_v[r, :] = jnp.ones((_LANES,), jnp.float32)
        return 0
    lax.fori_loop(0, _BLK, _ones, 0)

    row0 = sid * _CRPT
    pltpu.sync_copy(zcnt_v, cnt_sh.at[pl.ds(row0, _CRPT)])
    plsc.subcore_barrier()

    def _block(blk, _):
        base = wid * _EPW + blk * _BLK
        pltpu.sync_copy(dst_hbm.at[pl.ds(base, _BLK)], dst_v)
        pltpu.sync_copy(ones_v, cnt_sh.at[dst_v], add=True)
        return 0
    lax.fori_loop(0, _NBLK, _block, 0)

    plsc.subcore_barrier()
    pltpu.sync_copy(cnt_sh.at[pl.ds(row0, _CRPT)], cnt_hbm.at[cid, pl.ds(row0, _CRPT)])


_sc_cnt = pl.kernel(
    _sc_cnt_body,
    out_type=jax.ShapeDtypeStruct((_NC, _CPAD, _LANES), jnp.float32),
    mesh=plsc.VectorSubcoreMesh(**_MESH),
    scratch_types=[
        pltpu.VMEM_SHARED((_CPAD, _LANES), jnp.float32),
        pltpu.VMEM((_CRPT, _LANES), jnp.float32),
        pltpu.VMEM((_BLK,), jnp.int32),
        pltpu.VMEM((_BLK, _LANES), jnp.float32),
    ],
    compiler_params=pltpu.CompilerParams(use_tc_tiling_on_sc=False),
    name="genconv_sc_cnt",
)


_EB = 2000


def _edge_proj_body(ea_ref, w0_ref, w1_ref, w2_ref, e0_ref, e1_ref, e2_ref):
    a = ea_ref[...]
    e0_ref[...] = jnp.dot(a, w0_ref[...], preferred_element_type=jnp.float32)
    e1_ref[...] = jnp.dot(a, w1_ref[...], preferred_element_type=jnp.float32)
    e2_ref[...] = jnp.dot(a, w2_ref[...], preferred_element_type=jnp.float32)


def _edge_proj(edge_attr, We0, We1, We2):
    grid = (_E // _EB,)
    w_spec = pl.BlockSpec((_DE, _D), lambda i: (0, 0))
    e_spec = pl.BlockSpec((_EB, _D), lambda i: (i, 0))
    return pl.pallas_call(
        _edge_proj_body,
        grid=grid,
        in_specs=[pl.BlockSpec((_EB, _DE), lambda i: (i, 0)), w_spec, w_spec, w_spec],
        out_specs=[e_spec, e_spec, e_spec],
        out_shape=[jax.ShapeDtypeStruct((_E, _D), jnp.float32)] * 3,
    )(edge_attr, We0, We1, We2)


_NB = 2000


def _mlp_body(final, p0_ref, p1_ref, inv_ref, x_ref, w1_ref, w2_ref, g_ref,
              b_ref, wh_ref, bh_ref, o_ref):
    out = (p0_ref[0] + p1_ref[0]) * inv_ref[...] + x_ref[...]
    h = jnp.dot(out, w1_ref[...], preferred_element_type=jnp.float32)
    scale = g_ref[...] * (1.0 / jnp.sqrt(1.0 + _BN_EPS))
    h = jnp.maximum(h * scale + b_ref[...], 0.0)
    y = jnp.maximum(jnp.dot(h, w2_ref[...], preferred_element_type=jnp.float32), 0.0)
    if final:
        o_ref[...] = jnp.sum(y * wh_ref[...], axis=1, keepdims=True) + bh_ref[...]
    else:
        o_ref[...] = y


def _mlp(p, inv, x, W1, W2, gamma, beta, Wh, bh, final):
    import functools
    grid = (_N // _NB,)
    in_specs = [
        pl.BlockSpec((1, _NB, _D), lambda i: (0, i, 0)),
        pl.BlockSpec((1, _NB, _D), lambda i: (1, i, 0)),
        pl.BlockSpec((_NB, 1), lambda i: (i, 0)),
        pl.BlockSpec((_NB, _D), lambda i: (i, 0)),
        pl.BlockSpec((_D, _H), lambda i: (0, 0)),
        pl.BlockSpec((_H, _D), lambda i: (0, 0)),
        pl.BlockSpec((1, _H), lambda i: (0, 0)),
        pl.BlockSpec((1, _H), lambda i: (0, 0)),
        pl.BlockSpec((1, _D), lambda i: (0, 0)),
        pl.BlockSpec((1, 1), lambda i: (0, 0)),
    ]
    if final:
        out_spec = pl.BlockSpec((_NB, 1), lambda i: (i, 0))
        out_shape = jax.ShapeDtypeStruct((_N, 1), jnp.float32)
    else:
        out_spec = pl.BlockSpec((_NB, _D), lambda i: (i, 0))
        out_shape = jax.ShapeDtypeStruct((_N, _D), jnp.float32)
    return pl.pallas_call(
        functools.partial(_mlp_body, final),
        grid=grid,
        in_specs=in_specs,
        out_specs=out_spec,
        out_shape=out_shape,
    )(p, p, inv, x, W1, W2, gamma.reshape(1, _H), beta.reshape(1, _H),
      Wh.reshape(1, _D), bh.reshape(1, 1))


def kernel(x, edge_index, edge_attr, batch, num_graphs, graph_features,
           We0, W10, W20, gamma0, beta0,
           We1, W11, W21, gamma1, beta1,
           We2, W12, W22, gamma2, beta2,
           Wh, bh):
    src = edge_index[0]
    dst = edge_index[1]
    dst3 = dst.reshape(_NW, _NBLK, _BLK)
    zrows = jnp.zeros((_N, _D), jnp.float32)
    e0, e1, e2 = _edge_proj(edge_attr, We0, We1, We2)

    cnt2 = _sc_cnt(dst)
    cnt = cnt2[0, :_N, 0] + cnt2[1, :_N, 0]
    inv = (1.0 / jnp.maximum(cnt, 1.0)).reshape(_N, 1)

    p = _sc_msg(x, e0, zrows, src, dst3)
    h1 = _mlp(p, inv, x, W10, W20, gamma0, beta0, Wh, bh, final=False)
    p = _sc_msg(h1, e1, zrows, src, dst3)
    h2 = _mlp(p, inv, h1, W11, W21, gamma1, beta1, Wh, bh, final=False)
    p = _sc_msg(h2, e2, zrows, src, dst3)
    return _mlp(p, inv, h2, W12, W22, gamma2, beta2, Wh, bh, final=True)
